# Initial kernel scaffold; baseline (speedup 1.0000x reference)
import functools

import jax
import jax.numpy as jnp
from jax import lax
from jax.experimental import pallas as pl
from jax.experimental.pallas import tpu as pltpu
from jax.experimental.pallas import tpu_sc as plsc

VOCAB = 100002
EMB_DIM = 128
BATCH = 1024
SEQ = 200
PAD_IX = 0

_NC = 2
_NS = 16
_NW = _NC * _NS

_N = BATCH * SEQ
_BPW = _N // _NW
_CHUNK = 256
_NCHUNK = _BPW // _CHUNK
_LANES = 16


def _sc_body(table_hbm, idx_hbm, out_hbm, mask_hbm, idx_v, mask_v, rows_v, sem):
    wid = lax.axis_index("s") * _NC + lax.axis_index("c")
    base = wid * _BPW

    pltpu.sync_copy(idx_hbm.at[pl.ds(base, _BPW)], idx_v)

    def chunk_step(c, carry):
        off = c * _CHUNK
        pltpu.async_copy(table_hbm.at[idx_v.at[pl.ds(off, _CHUNK)]], rows_v, sem).wait()
        pltpu.sync_copy(rows_v, out_hbm.at[pl.ds(base + off, _CHUNK)])
        return carry

    lax.fori_loop(0, _NCHUNK, chunk_step, 0)

    def mask_step(i, carry):
        v = idx_v[pl.ds(i * _LANES, _LANES)]
        mask_v[pl.ds(i * _LANES, _LANES)] = (v != PAD_IX).astype(jnp.int32)
        return carry

    lax.fori_loop(0, _BPW // _LANES, mask_step, 0)
    pltpu.sync_copy(mask_v, mask_hbm.at[pl.ds(base, _BPW)])


@jax.jit
def _sc_embed(table, idx):
    mesh = plsc.VectorSubcoreMesh(core_axis_name="c", subcore_axis_name="s")
    f = functools.partial(
        pl.kernel,
        out_type=(
            jax.ShapeDtypeStruct((_N, EMB_DIM), jnp.float32),
            jax.ShapeDtypeStruct((_N,), jnp.int32),
        ),
        mesh=mesh,
        scratch_types=[
            pltpu.VMEM((_BPW,), jnp.int32),
            pltpu.VMEM((_BPW,), jnp.int32),
            pltpu.VMEM((_CHUNK, EMB_DIM), jnp.float32),
            pltpu.SemaphoreType.DMA,
        ],
    )(_sc_body)
    return f(table, idx)


def kernel(encoded, table):
    idx = encoded.reshape(_N)
    out_flat, mask_flat = _sc_embed(table, idx)
    return (
        out_flat.reshape(BATCH, SEQ, EMB_DIM),
        mask_flat.reshape(BATCH, SEQ),
        encoded,
    )


# speedup vs baseline: 6.3975x; 6.3975x over previous
import functools

import jax
import jax.numpy as jnp
from jax import lax
from jax.experimental import pallas as pl
from jax.experimental.pallas import tpu as pltpu
from jax.experimental.pallas import tpu_sc as plsc

VOCAB = 100002
EMB_DIM = 128
BATCH = 1024
SEQ = 200
PAD_IX = 0

_NC = 2
_NS = 16
_NW = _NC * _NS

_N = BATCH * SEQ
_BPW = _N // _NW
_CHUNK = 256
_NCHUNK = _BPW // _CHUNK
_LANES = 16


def _sc_body(table_hbm, idx_hbm, out_hbm, mask_hbm, idx_v, mask_v, rows_v, sem):
    wid = lax.axis_index("s") * _NC + lax.axis_index("c")
    base = wid * _BPW

    pltpu.sync_copy(idx_hbm.at[pl.ds(base, _BPW)], idx_v)

    def chunk_step(c, carry):
        off = c * _CHUNK
        pltpu.async_copy(table_hbm.at[idx_v.at[pl.ds(off, _CHUNK)]], rows_v, sem).wait()
        pltpu.sync_copy(rows_v, out_hbm.at[pl.ds(base + off, _CHUNK)])
        return carry

    lax.fori_loop(0, _NCHUNK, chunk_step, 0)

    def mask_step(i, carry):
        v = idx_v[pl.ds(i * _LANES, _LANES)]
        mask_v[pl.ds(i * _LANES, _LANES)] = jnp.minimum(v, 1)
        return carry

    lax.fori_loop(0, _BPW // _LANES, mask_step, 0)
    pltpu.sync_copy(mask_v, mask_hbm.at[pl.ds(base, _BPW)])


@jax.jit
def _sc_embed(table, idx):
    mesh = plsc.VectorSubcoreMesh(core_axis_name="c", subcore_axis_name="s")
    f = functools.partial(
        pl.kernel,
        out_type=(
            jax.ShapeDtypeStruct((_N, EMB_DIM), jnp.float32),
            jax.ShapeDtypeStruct((_N,), jnp.int32),
        ),
        mesh=mesh,
        scratch_types=[
            pltpu.VMEM((_BPW,), jnp.int32),
            pltpu.VMEM((_BPW,), jnp.int32),
            pltpu.VMEM((_CHUNK, EMB_DIM), jnp.float32),
            pltpu.SemaphoreType.DMA,
        ],
    )(_sc_body)
    return f(table, idx)


def kernel(encoded, table):
    idx = encoded.reshape(_N)
    out_flat, mask_flat = _sc_embed(table, idx)
    return (
        out_flat.reshape(BATCH, SEQ, EMB_DIM),
        mask_flat.reshape(BATCH, SEQ),
        encoded,
    )


# speedup vs baseline: 7.5519x; 1.1804x over previous
import functools

import jax
import jax.numpy as jnp
from jax import lax
from jax.experimental import pallas as pl
from jax.experimental.pallas import tpu as pltpu
from jax.experimental.pallas import tpu_sc as plsc

VOCAB = 100002
EMB_DIM = 128
BATCH = 1024
SEQ = 200
PAD_IX = 0

_NC = 2
_NS = 16
_NW = _NC * _NS

_N = BATCH * SEQ
_BPW = _N // _NW
_CHUNK = 320
_NCHUNK = _BPW // _CHUNK
_NPAIR = _NCHUNK // 2
_LANES = 16


def _sc_body(table_hbm, idx_hbm, out_hbm, mask_hbm,
             idx_v, mask_v, rows0, rows1, gsem0, gsem1):
    wid = lax.axis_index("s") * _NC + lax.axis_index("c")
    base = wid * _BPW

    pltpu.sync_copy(idx_hbm.at[pl.ds(base, _BPW)], idx_v)

    rows = (rows0, rows1)
    gsem = (gsem0, gsem1)

    def g_desc(c, b):
        return pltpu.make_async_copy(
            table_hbm.at[idx_v.at[pl.ds(c * _CHUNK, _CHUNK)]], rows[b], gsem[b])

    def wb(c, b):
        pltpu.sync_copy(rows[b], out_hbm.at[pl.ds(base + c * _CHUNK, _CHUNK)])

    g_desc(0, 0).start()
    g_desc(1, 1).start()

    def mask_step(i, carry):
        o = i * (4 * _LANES)
        for j in range(4):
            v = idx_v[pl.ds(o + j * _LANES, _LANES)]
            mask_v[pl.ds(o + j * _LANES, _LANES)] = jnp.minimum(v, 1)
        return carry

    lax.fori_loop(0, _BPW // (4 * _LANES), mask_step, 0)
    pltpu.sync_copy(mask_v, mask_hbm.at[pl.ds(base, _BPW)])

    def pair_body(p, carry):
        c0 = p * 2
        g_desc(c0, 0).wait()
        wb(c0, 0)
        g_desc(c0 + 2, 0).start()
        g_desc(c0 + 1, 1).wait()
        wb(c0 + 1, 1)
        g_desc(c0 + 3, 1).start()
        return carry

    lax.fori_loop(0, _NPAIR - 1, pair_body, 0)

    c0 = _NCHUNK - 2
    g_desc(c0, 0).wait()
    wb(c0, 0)
    g_desc(c0 + 1, 1).wait()
    wb(c0 + 1, 1)


@jax.jit
def _sc_embed(table, idx):
    mesh = plsc.VectorSubcoreMesh(core_axis_name="c", subcore_axis_name="s")
    f = functools.partial(
        pl.kernel,
        out_type=(
            jax.ShapeDtypeStruct((_N, EMB_DIM), jnp.float32),
            jax.ShapeDtypeStruct((_N,), jnp.int32),
        ),
        mesh=mesh,
        scratch_types=[
            pltpu.VMEM((_BPW,), jnp.int32),
            pltpu.VMEM((_BPW,), jnp.int32),
            pltpu.VMEM((_CHUNK, EMB_DIM), jnp.float32),
            pltpu.VMEM((_CHUNK, EMB_DIM), jnp.float32),
            pltpu.SemaphoreType.DMA,
            pltpu.SemaphoreType.DMA,
        ],
    )(_sc_body)
    return f(table, idx)


def kernel(encoded, table):
    idx = encoded.reshape(_N)
    out_flat, mask_flat = _sc_embed(table, idx)
    return (
        out_flat.reshape(BATCH, SEQ, EMB_DIM),
        mask_flat.reshape(BATCH, SEQ),
        encoded,
    )


# speedup vs baseline: 7.5831x; 1.0041x over previous
import functools

import jax
import jax.numpy as jnp
from jax import lax
from jax.experimental import pallas as pl
from jax.experimental.pallas import tpu as pltpu
from jax.experimental.pallas import tpu_sc as plsc

VOCAB = 100002
EMB_DIM = 128
BATCH = 1024
SEQ = 200
PAD_IX = 0

_NC = 2
_NS = 16
_NW = _NC * _NS

_N = BATCH * SEQ
_BPW = _N // _NW
_CHUNK = 160
_NCHUNK = _BPW // _CHUNK
_NBUF = 4
_NGROUP = _NCHUNK // _NBUF
_LANES = 16


def _sc_body(table_hbm, idx_hbm, out_hbm, mask_hbm,
             idx_v, mask_v, rows0, rows1, rows2, rows3,
             gsem0, gsem1, gsem2, gsem3):
    wid = lax.axis_index("s") * _NC + lax.axis_index("c")
    base = wid * _BPW

    pltpu.sync_copy(idx_hbm.at[pl.ds(base, _BPW)], idx_v)

    rows = (rows0, rows1, rows2, rows3)
    gsem = (gsem0, gsem1, gsem2, gsem3)

    def g_desc(c, b):
        return pltpu.make_async_copy(
            table_hbm.at[idx_v.at[pl.ds(c * _CHUNK, _CHUNK)]], rows[b], gsem[b])

    def wb(c, b):
        pltpu.sync_copy(rows[b], out_hbm.at[pl.ds(base + c * _CHUNK, _CHUNK)])

    for b in range(_NBUF):
        g_desc(b, b).start()

    def mask_step(i, carry):
        o = i * (4 * _LANES)
        for j in range(4):
            v = idx_v[pl.ds(o + j * _LANES, _LANES)]
            mask_v[pl.ds(o + j * _LANES, _LANES)] = jnp.minimum(v, 1)
        return carry

    lax.fori_loop(0, _BPW // (4 * _LANES), mask_step, 0)
    pltpu.sync_copy(mask_v, mask_hbm.at[pl.ds(base, _BPW)])

    def group_body(p, carry):
        c0 = p * _NBUF
        for b in range(_NBUF):
            g_desc(c0 + b, b).wait()
            wb(c0 + b, b)
            g_desc(c0 + b + _NBUF, b).start()
        return carry

    lax.fori_loop(0, _NGROUP - 1, group_body, 0)

    c0 = (_NGROUP - 1) * _NBUF
    for b in range(_NBUF):
        g_desc(c0 + b, b).wait()
        wb(c0 + b, b)


@jax.jit
def _sc_embed(table, idx):
    mesh = plsc.VectorSubcoreMesh(core_axis_name="c", subcore_axis_name="s")
    f = functools.partial(
        pl.kernel,
        out_type=(
            jax.ShapeDtypeStruct((_N, EMB_DIM), jnp.float32),
            jax.ShapeDtypeStruct((_N,), jnp.int32),
        ),
        mesh=mesh,
        scratch_types=[
            pltpu.VMEM((_BPW,), jnp.int32),
            pltpu.VMEM((_BPW,), jnp.int32),
            pltpu.VMEM((_CHUNK, EMB_DIM), jnp.float32),
            pltpu.VMEM((_CHUNK, EMB_DIM), jnp.float32),
            pltpu.VMEM((_CHUNK, EMB_DIM), jnp.float32),
            pltpu.VMEM((_CHUNK, EMB_DIM), jnp.float32),
            pltpu.SemaphoreType.DMA,
            pltpu.SemaphoreType.DMA,
            pltpu.SemaphoreType.DMA,
            pltpu.SemaphoreType.DMA,
        ],
    )(_sc_body)
    return f(table, idx)


def kernel(encoded, table):
    idx = encoded.reshape(_N)
    out_flat, mask_flat = _sc_embed(table, idx)
    return (
        out_flat.reshape(BATCH, SEQ, EMB_DIM),
        mask_flat.reshape(BATCH, SEQ),
        encoded,
    )


# speedup vs baseline: 7.8546x; 1.0358x over previous
import functools

import jax
import jax.numpy as jnp
from jax import lax
from jax.experimental import pallas as pl
from jax.experimental.pallas import tpu as pltpu
from jax.experimental.pallas import tpu_sc as plsc

VOCAB = 100002
EMB_DIM = 128
BATCH = 1024
SEQ = 200
PAD_IX = 0

_NC = 2
_NS = 16
_NW = _NC * _NS

_N = BATCH * SEQ
_BPW = _N // _NW
_CHUNK = 160
_NCHUNK = _BPW // _CHUNK
_NBUF = 4
_NGROUP = _NCHUNK // _NBUF


def _sc_body(table_hbm, idx_hbm, out_hbm,
             idx_v, rows0, rows1, rows2, rows3,
             gsem0, gsem1, gsem2, gsem3):
    wid = lax.axis_index("s") * _NC + lax.axis_index("c")
    base = wid * _BPW

    pltpu.sync_copy(idx_hbm.at[pl.ds(base, _BPW)], idx_v)

    rows = (rows0, rows1, rows2, rows3)
    gsem = (gsem0, gsem1, gsem2, gsem3)

    def g_desc(c, b):
        return pltpu.make_async_copy(
            table_hbm.at[idx_v.at[pl.ds(c * _CHUNK, _CHUNK)]], rows[b], gsem[b])

    def wb(c, b):
        pltpu.sync_copy(rows[b], out_hbm.at[pl.ds(base + c * _CHUNK, _CHUNK)])

    for b in range(_NBUF):
        g_desc(b, b).start()

    def group_body(p, carry):
        c0 = p * _NBUF
        for b in range(_NBUF):
            g_desc(c0 + b, b).wait()
            wb(c0 + b, b)
            g_desc(c0 + b + _NBUF, b).start()
        return carry

    lax.fori_loop(0, _NGROUP - 1, group_body, 0)

    c0 = (_NGROUP - 1) * _NBUF
    for b in range(_NBUF):
        g_desc(c0 + b, b).wait()
        wb(c0 + b, b)


def _mask_body(enc_ref, mask_ref):
    mask_ref[...] = jnp.where(enc_ref[...] != PAD_IX, 1, 0).astype(jnp.int32)


@jax.jit
def _embed(table, idx, encoded):
    mesh = plsc.VectorSubcoreMesh(core_axis_name="c", subcore_axis_name="s")
    sc = functools.partial(
        pl.kernel,
        out_type=jax.ShapeDtypeStruct((_N, EMB_DIM), jnp.float32),
        mesh=mesh,
        scratch_types=[
            pltpu.VMEM((_BPW,), jnp.int32),
            pltpu.VMEM((_CHUNK, EMB_DIM), jnp.float32),
            pltpu.VMEM((_CHUNK, EMB_DIM), jnp.float32),
            pltpu.VMEM((_CHUNK, EMB_DIM), jnp.float32),
            pltpu.VMEM((_CHUNK, EMB_DIM), jnp.float32),
            pltpu.SemaphoreType.DMA,
            pltpu.SemaphoreType.DMA,
            pltpu.SemaphoreType.DMA,
            pltpu.SemaphoreType.DMA,
        ],
    )(_sc_body)
    out_flat = sc(table, idx)
    mask = pl.pallas_call(
        _mask_body,
        out_shape=jax.ShapeDtypeStruct((BATCH, SEQ), jnp.int32),
    )(encoded)
    return out_flat, mask


def kernel(encoded, table):
    idx = encoded.reshape(_N)
    out_flat, mask = _embed(table, idx, encoded)
    return out_flat.reshape(BATCH, SEQ, EMB_DIM), mask, encoded
